# baseline (speedup 1.0000x reference)
import functools

import jax
import jax.numpy as jnp
from jax import lax
from jax.experimental import pallas as pl
from jax.experimental.pallas import tpu as pltpu
from jax.experimental.pallas import tpu_sc as plsc

_TP = 256
_NC = 2
_NS = 16
_NW = _NC * _NS
_RC = 8


def _cnt_body(n_pass, a_ref, o_ref):
    a = a_ref[0]
    F = a.shape[0]
    iota = lax.broadcasted_iota(jnp.int32, a.shape, 0)
    work = a
    cnt = jnp.zeros(a.shape, jnp.int32)
    for j in range(n_pass):
        m = jnp.max(work, axis=0, keepdims=True)
        cnt = cnt + (a < m).astype(jnp.int32)
        if j < n_pass - 1:
            kidx = jnp.min(jnp.where(work < m, F, iota), axis=0, keepdims=True)
            work = jnp.where(iota == kidx, -jnp.inf, work)
    o_ref[0] = cnt


def _rank_counts(a3, n_pass):
    B, F, P = a3.shape
    tp = min(_TP, P)
    return pl.pallas_call(
        functools.partial(_cnt_body, n_pass),
        grid=(B, P // tp),
        in_specs=[pl.BlockSpec((1, F, tp), lambda b, p: (b, 0, p))],
        out_specs=pl.BlockSpec((1, F, tp), lambda b, p: (b, 0, p)),
        out_shape=jax.ShapeDtypeStruct((B, F, P), jnp.int32),
    )(a3)


def _sc_gather(cnt2, wext, F, n_ext):
    R, P = cnt2.shape
    rows_per_w = R // _NW
    n_chunks = rows_per_w // _RC
    nv = P // 16
    mesh = plsc.VectorSubcoreMesh(core_axis_name="c", subcore_axis_name="s")

    @functools.partial(
        pl.kernel,
        out_type=jax.ShapeDtypeStruct((R, P), jnp.float32),
        mesh=mesh,
        compiler_params=pltpu.CompilerParams(needs_layout_passes=False),
        scratch_types=[
            pltpu.VMEM((F * n_ext,), jnp.float32),
            pltpu.VMEM((_RC, P), jnp.int32),
            pltpu.VMEM((_RC, P), jnp.float32),
        ],
    )
    def sck(cnt_hbm, wext_hbm, out_hbm, w_v, c_v, o_v):
        wid = lax.axis_index("s") * _NC + lax.axis_index("c")
        pltpu.sync_copy(wext_hbm, w_v)
        base_row = wid * rows_per_w

        def chunk_body(ci, _):
            r0 = base_row + ci * _RC
            pltpu.sync_copy(cnt_hbm.at[pl.ds(r0, _RC)], c_v)

            def row_body(r, _):
                fbase = ((r0 + r) % F) * n_ext

                def vec_body(v, _):
                    c = c_v[r, pl.ds(v * 16, 16)]
                    w = plsc.load_gather(w_v, [c + fbase])
                    o_v[r, pl.ds(v * 16, 16)] = w
                    return _

                return lax.fori_loop(0, nv, vec_body, _)

            lax.fori_loop(0, _RC, row_body, 0)
            pltpu.sync_copy(o_v, out_hbm.at[pl.ds(r0, _RC)])
            return _

        lax.fori_loop(0, n_chunks, chunk_body, 0)

    return sck(cnt2, wext)


def kernel(activations, rank_weights):
    B, F, H, W = activations.shape
    n_pass = rank_weights.shape[1]
    P = H * W
    a3 = activations.reshape(B, F, P)
    cnt = _rank_counts(a3, n_pass)
    n_ext = n_pass + 1
    wext = jnp.concatenate(
        [rank_weights, jnp.zeros((F, 1), jnp.float32)], axis=1
    ).reshape(-1)
    out = _sc_gather(cnt.reshape(B * F, P), wext, F, n_ext)
    return out.reshape(B, F, H, W)
